# baseline (speedup 1.0000x reference)
import functools

import jax
import jax.numpy as jnp
import numpy as np
from jax import lax
from jax.experimental import pallas as pl
from jax.experimental.pallas import tpu as pltpu
from jax.experimental.pallas import tpu_sc as plsc

N = 10000
E = 320000
D = 128
H = D // 2
BETA = 1.0
EPS = 1e-07

NUM_SC = 2
NUM_SUB = 16
CHUNK = 128
NCHUNKS = E // CHUNK
N_PAD = 10240
ROWS_PER_SUB = N_PAD // NUM_SUB
DUMP_ROWS = 128
DUMPS_PER_SUB = ROWS_PER_SUB // DUMP_ROWS


def _sc_segment_sums(xr, comb2, idx_all):
    mesh = plsc.VectorSubcoreMesh(core_axis_name="c", subcore_axis_name="s")

    @functools.partial(
        pl.kernel,
        mesh=mesh,
        compiler_params=pltpu.CompilerParams(use_tc_tiling_on_sc=False),
        out_type=[
            jax.ShapeDtypeStruct((NUM_SC, N_PAD, H), jnp.float32),
            jax.ShapeDtypeStruct((NUM_SC, N_PAD, H), jnp.float32),
        ],
        scratch_types=[
            pltpu.VMEM((2, 6, CHUNK), jnp.int32),
            pltpu.VMEM((2, 3, CHUNK), jnp.int32),
            pltpu.VMEM((2, CHUNK, H), jnp.float32),
            pltpu.VMEM((2, CHUNK, H), jnp.float32),
            pltpu.VMEM((2, CHUNK, H // 2), jnp.int32),
            pltpu.VMEM_SHARED((N_PAD, H), jnp.float32),
            pltpu.VMEM_SHARED((N_PAD, H), jnp.float32),
            pltpu.SemaphoreType.DMA,
            pltpu.SemaphoreType.DMA,
            pltpu.SemaphoreType.DMA,
            pltpu.SemaphoreType.DMA,
            pltpu.SemaphoreType.DMA,
            pltpu.SemaphoreType.DMA,
        ],
    )
    def k(xr_hbm, comb_hbm, idx_hbm,
          s1_out, s2_out,
          idxb, gidx, xrows, mex, cb, s1_sp, s2_sp,
          idxsem0, idxsem1, gsem0, gsem1, scatsem0, scatsem1):
        c = lax.axis_index("c")
        s = lax.axis_index("s")
        idxsem = (idxsem0, idxsem1)
        gsem = (gsem0, gsem1)
        scatsem = (scatsem0, scatsem1)

        def zrow(r, carry):
            for g in range(H // 16):
                xrows[0, r, pl.ds(16 * g, 16)] = jnp.zeros((16,), jnp.float32)
            return carry
        lax.fori_loop(0, DUMP_ROWS, zrow, 0)

        def zstripe(k, carry):
            off = s * ROWS_PER_SUB + k * DUMP_ROWS
            pltpu.sync_copy(xrows.at[0], s1_sp.at[pl.ds(off, DUMP_ROWS)])
            pltpu.sync_copy(xrows.at[0], s2_sp.at[pl.ds(off, DUMP_ROWS)])
            return carry
        lax.fori_loop(0, DUMPS_PER_SUB, zstripe, 0)
        plsc.subcore_barrier()

        NUNIF = (NCHUNKS // NUM_SUB) & ~1

        def idx_start(b, j):
            return pltpu.async_copy(idx_hbm.at[s + j * NUM_SUB], idxb.at[b],
                                    idxsem[b])

        def idx_wait(b):
            pltpu.make_async_copy(idx_hbm.at[s], idxb.at[b], idxsem[b]).wait()

        def idxcomp(b):
            for g in range(CHUNK // 16):
                sl = pl.ds(16 * g, 16)
                cidx = ((idxb[b, 2, sl] * 8 + idxb[b, 3, sl]) * 8
                        + idxb[b, 4, sl]) * 8 + idxb[b, 5, sl]
                gidx[b, 1, sl] = cidx * 2 + c
                gidx[b, 0, sl] = idxb[b, 0, sl] * 2 + c
                gidx[b, 2, sl] = idxb[b, 1, sl]

        def gather_start(b):
            pltpu.async_copy(xr_hbm.at[gidx.at[b, 0]], xrows.at[b], gsem[b])
            pltpu.async_copy(comb_hbm.at[gidx.at[b, 1]], cb.at[b], gsem[b])

        def gather_wait(b):
            pltpu.make_async_copy(xr_hbm.at[gidx.at[b, 0]], xrows.at[b],
                                  gsem[b]).wait()
            pltpu.make_async_copy(comb_hbm.at[gidx.at[b, 1]], cb.at[b],
                                  gsem[b]).wait()

        def compute(b):
            def row(ri, inner):
                for u in range(2):
                    r = 2 * ri + u
                    for q in range(H // 32):
                        bits = cb[b, r, pl.ds(16 * q, 16)]
                        emb0 = lax.bitcast_convert_type(bits << 16, jnp.float32)
                        emb1 = lax.bitcast_convert_type(
                            bits & jnp.int32(-65536), jnp.float32)
                        for h, emb in ((0, emb0), (1, emb1)):
                            sl = pl.ds(32 * q + 16 * h, 16)
                            m = (jnp.maximum(xrows[b, r, sl] + emb, 0.0)
                                 + EPS)
                            ex = jnp.exp(BETA * m)
                            xrows[b, r, sl] = ex
                            mex[b, r, sl] = m * ex
                return inner
            lax.fori_loop(0, CHUNK // 2, row, 0)

        def scat_start(b):
            pltpu.async_copy(xrows.at[b], s1_sp.at[gidx.at[b, 2]], scatsem[b],
                             add=True)
            pltpu.async_copy(mex.at[b], s2_sp.at[gidx.at[b, 2]], scatsem[b],
                             add=True)

        def scat_wait(b):
            pltpu.make_async_copy(xrows.at[b], s1_sp.at[gidx.at[b, 2]],
                                  scatsem[b]).wait()
            pltpu.make_async_copy(mex.at[b], s2_sp.at[gidx.at[b, 2]],
                                  scatsem[b]).wait()

        def process_sync(b):
            compute(b)
            pltpu.sync_copy(xrows.at[b], s1_sp.at[gidx.at[b, 2]], add=True)
            pltpu.sync_copy(mex.at[b], s2_sp.at[gidx.at[b, 2]], add=True)

        idx_start(0, 0)
        idx_start(1, 1)

        def pair(t, carry):
            for b in (0, 1):
                j = 2 * t + b
                idx_wait(b)
                @pl.when(t > 0)
                def _():
                    scat_wait(b)
                idxcomp(b)

                @pl.when(t < NUNIF // 2 - 1)
                def _():
                    idx_start(b, j + 2)
                gather_start(b)
                b1 = 1 - b
                if b == 0:
                    @pl.when(t > 0)
                    def _():
                        gather_wait(b1)
                        compute(b1)
                        scat_start(b1)
                else:
                    gather_wait(b1)
                    compute(b1)
                    scat_start(b1)
            return carry
        lax.fori_loop(0, NUNIF // 2, pair, 0)
        gather_wait(1)
        process_sync(1)
        scat_wait(0)

        ntail = NCHUNKS - NUNIF * NUM_SUB

        @pl.when(s < ntail)
        def _():
            pltpu.sync_copy(idx_hbm.at[NUNIF * NUM_SUB + s], idxb.at[0])
            idxcomp(0)
            gather_start(0)
            gather_wait(0)
            process_sync(0)

        plsc.subcore_barrier()

        def dstripe(k, carry):
            off = s * ROWS_PER_SUB + k * DUMP_ROWS
            pltpu.sync_copy(s1_sp.at[pl.ds(off, DUMP_ROWS)], xrows.at[0])
            pltpu.sync_copy(xrows.at[0], s1_out.at[c, pl.ds(off, DUMP_ROWS)])
            pltpu.sync_copy(s2_sp.at[pl.ds(off, DUMP_ROWS)], mex.at[0])
            pltpu.sync_copy(mex.at[0], s2_out.at[c, pl.ds(off, DUMP_ROWS)])
            return carry
        lax.fori_loop(0, DUMPS_PER_SUB, dstripe, 0)

    return k(xr, comb2, idx_all)


_BR = 1000


def _tc_body(x_ref, s1_ref, s2_ref, w_ref, b_ref, o_ref):
    s1 = s1_ref[...]
    s2 = s2_ref[...]
    agg = s2 / (s1 + 1e-16)
    aggf = jnp.concatenate([agg[0], agg[1]], axis=-1)
    feats = x_ref[...] + aggf
    o_ref[...] = (jnp.dot(feats, w_ref[...], preferred_element_type=jnp.float32)
                  + b_ref[0])


def _tc_finish(x, s1, s2, W, b):
    grid = (N // _BR,)
    return pl.pallas_call(
        _tc_body,
        grid=grid,
        in_specs=[
            pl.BlockSpec((_BR, D), lambda i: (i, 0)),
            pl.BlockSpec((NUM_SC, _BR, H), lambda i: (0, i, 0)),
            pl.BlockSpec((NUM_SC, _BR, H), lambda i: (0, i, 0)),
            pl.BlockSpec((D, D), lambda i: (0, 0)),
            pl.BlockSpec((1, D), lambda i: (0, 0)),
        ],
        out_specs=pl.BlockSpec((_BR, D), lambda i: (i, 0)),
        out_shape=jax.ShapeDtypeStruct((N, D), jnp.float32),
    )(x, s1, s2, W, b.reshape(1, D))


def kernel(x, edge_index, edge_attr, bond_tables, W, b):
    src = edge_index[0]
    dst = edge_index[1]
    idx_all = jnp.stack(
        [src.reshape(NCHUNKS, CHUNK), dst.reshape(NCHUNKS, CHUNK)]
        + [edge_attr[:, f].reshape(NCHUNKS, CHUNK) for f in range(4)],
        axis=1)
    t0, t1, t2, t3 = bond_tables[0], bond_tables[1], bond_tables[2], bond_tables[3]
    comb = (t0[:, None, None, None, :] + t1[None, :, None, None, :]
            + t2[None, None, :, None, :] + t3[None, None, None, :, :]
            ).reshape(8 * 8 * 8 * 8, D)
    xr = x.reshape(2 * N, H)
    comb2 = comb.reshape(2 * 8 * 8 * 8 * 8, H)
    comb2 = (comb2.reshape(-1, H // 32, 2, 16).transpose(0, 1, 3, 2)
             .astype(jnp.bfloat16))
    comb2 = lax.bitcast_convert_type(comb2.reshape(-1, H // 2, 2), jnp.int32)
    s1, s2 = _sc_segment_sums(xr, comb2, idx_all)
    return _tc_finish(x, s1, s2, W, b)


# speedup vs baseline: 2.4487x; 2.4487x over previous
import functools

import jax
import jax.numpy as jnp
import numpy as np
from jax import lax
from jax.experimental import pallas as pl
from jax.experimental.pallas import tpu as pltpu
from jax.experimental.pallas import tpu_sc as plsc

N = 10000
E = 320000
D = 128
H = D // 2
BETA = 1.0
EPS = 1e-07

NUM_SC = 2
NUM_SUB = 16
CHUNK = 128
NCHUNKS = E // CHUNK
N_PAD = 10240
ROWS_PER_SUB = N_PAD // NUM_SUB
DUMP_ROWS = 128
DUMPS_PER_SUB = ROWS_PER_SUB // DUMP_ROWS


def _sc_segment_sums(xr, comb2, idx_all):
    mesh = plsc.VectorSubcoreMesh(core_axis_name="c", subcore_axis_name="s")

    @functools.partial(
        pl.kernel,
        mesh=mesh,
        compiler_params=pltpu.CompilerParams(use_tc_tiling_on_sc=False),
        out_type=[
            jax.ShapeDtypeStruct((NUM_SC, N_PAD, H), jnp.float32),
            jax.ShapeDtypeStruct((NUM_SC, N_PAD, H), jnp.float32),
        ],
        scratch_types=[
            pltpu.VMEM((2, 6, CHUNK), jnp.int32),
            pltpu.VMEM((2, 3, CHUNK), jnp.int32),
            pltpu.VMEM((2, CHUNK, H), jnp.float32),
            pltpu.VMEM((2, CHUNK, H), jnp.float32),
            pltpu.VMEM_SHARED((N_PAD, H), jnp.float32),
            pltpu.VMEM_SHARED((N_PAD, H), jnp.float32),
            pltpu.SemaphoreType.DMA,
            pltpu.SemaphoreType.DMA,
            pltpu.SemaphoreType.DMA,
            pltpu.SemaphoreType.DMA,
            pltpu.SemaphoreType.DMA,
            pltpu.SemaphoreType.DMA,
        ],
    )
    def k(xr_hbm, comb_hbm, idx_hbm,
          s1_out, s2_out,
          idxb, gidx, xrows, crows, s1_sp, s2_sp,
          idxsem0, idxsem1, gsem0, gsem1, scatsem0, scatsem1):
        c = lax.axis_index("c")
        s = lax.axis_index("s")
        idxsem = (idxsem0, idxsem1)
        gsem = (gsem0, gsem1)
        scatsem = (scatsem0, scatsem1)

        def zrow(r, carry):
            for g in range(H // 16):
                xrows[0, r, pl.ds(16 * g, 16)] = jnp.zeros((16,), jnp.float32)
            return carry
        lax.fori_loop(0, DUMP_ROWS, zrow, 0)

        def zstripe(k, carry):
            off = s * ROWS_PER_SUB + k * DUMP_ROWS
            pltpu.sync_copy(xrows.at[0], s1_sp.at[pl.ds(off, DUMP_ROWS)])
            pltpu.sync_copy(xrows.at[0], s2_sp.at[pl.ds(off, DUMP_ROWS)])
            return carry
        lax.fori_loop(0, DUMPS_PER_SUB, zstripe, 0)
        plsc.subcore_barrier()

        NUNIF = (NCHUNKS // NUM_SUB) & ~1

        def idx_start(b, j):
            return pltpu.async_copy(idx_hbm.at[s + j * NUM_SUB], idxb.at[b],
                                    idxsem[b])

        def idx_wait(b):
            pltpu.make_async_copy(idx_hbm.at[s], idxb.at[b], idxsem[b]).wait()

        def idxcomp(b):
            for g in range(CHUNK // 16):
                sl = pl.ds(16 * g, 16)
                cidx = ((idxb[b, 2, sl] * 8 + idxb[b, 3, sl]) * 8
                        + idxb[b, 4, sl]) * 8 + idxb[b, 5, sl]
                gidx[b, 1, sl] = cidx * 2 + c
                gidx[b, 0, sl] = idxb[b, 0, sl] * 2 + c
                gidx[b, 2, sl] = idxb[b, 1, sl]

        def gather_start(b):
            pltpu.async_copy(xr_hbm.at[gidx.at[b, 0]], xrows.at[b], gsem[b])
            pltpu.async_copy(comb_hbm.at[gidx.at[b, 1]], crows.at[b], gsem[b])

        def gather_wait(b):
            pltpu.make_async_copy(xr_hbm.at[gidx.at[b, 0]], xrows.at[b],
                                  gsem[b]).wait()
            pltpu.make_async_copy(comb_hbm.at[gidx.at[b, 1]], crows.at[b],
                                  gsem[b]).wait()

        def compute(b):
            def row(ri, inner):
                for u in range(2):
                    r = 2 * ri + u
                    for g in range(H // 16):
                        sl = pl.ds(16 * g, 16)
                        m = (jnp.maximum(xrows[b, r, sl] + crows[b, r, sl], 0.0)
                             + EPS)
                        ex = jnp.exp(BETA * m)
                        xrows[b, r, sl] = ex
                        crows[b, r, sl] = m * ex
                return inner
            lax.fori_loop(0, CHUNK // 2, row, 0)

        def scat_start(b):
            pltpu.async_copy(xrows.at[b], s1_sp.at[gidx.at[b, 2]], scatsem[b],
                             add=True)
            pltpu.async_copy(crows.at[b], s2_sp.at[gidx.at[b, 2]], scatsem[b],
                             add=True)

        def scat_wait(b):
            pltpu.make_async_copy(xrows.at[b], s1_sp.at[gidx.at[b, 2]],
                                  scatsem[b]).wait()
            pltpu.make_async_copy(crows.at[b], s2_sp.at[gidx.at[b, 2]],
                                  scatsem[b]).wait()

        def process_sync(b):
            compute(b)
            pltpu.sync_copy(xrows.at[b], s1_sp.at[gidx.at[b, 2]], add=True)
            pltpu.sync_copy(crows.at[b], s2_sp.at[gidx.at[b, 2]], add=True)

        idx_start(0, 0)
        idx_start(1, 1)

        def pair(t, carry):
            for b in (0, 1):
                j = 2 * t + b
                idx_wait(b)
                @pl.when(t > 0)
                def _():
                    scat_wait(b)
                idxcomp(b)

                @pl.when(t < NUNIF // 2 - 1)
                def _():
                    idx_start(b, j + 2)
                gather_start(b)
                b1 = 1 - b
                if b == 0:
                    @pl.when(t > 0)
                    def _():
                        gather_wait(b1)
                        compute(b1)
                        scat_start(b1)
                else:
                    gather_wait(b1)
                    compute(b1)
                    scat_start(b1)
            return carry
        lax.fori_loop(0, NUNIF // 2, pair, 0)
        gather_wait(1)
        process_sync(1)
        scat_wait(0)

        ntail = NCHUNKS - NUNIF * NUM_SUB

        @pl.when(s < ntail)
        def _():
            pltpu.sync_copy(idx_hbm.at[NUNIF * NUM_SUB + s], idxb.at[0])
            idxcomp(0)
            gather_start(0)
            gather_wait(0)
            process_sync(0)

        plsc.subcore_barrier()

        def dstripe(k, carry):
            off = s * ROWS_PER_SUB + k * DUMP_ROWS
            pltpu.sync_copy(s1_sp.at[pl.ds(off, DUMP_ROWS)], xrows.at[0])
            pltpu.sync_copy(xrows.at[0], s1_out.at[c, pl.ds(off, DUMP_ROWS)])
            pltpu.sync_copy(s2_sp.at[pl.ds(off, DUMP_ROWS)], crows.at[0])
            pltpu.sync_copy(crows.at[0], s2_out.at[c, pl.ds(off, DUMP_ROWS)])
            return carry
        lax.fori_loop(0, DUMPS_PER_SUB, dstripe, 0)

    return k(xr, comb2, idx_all)


_BR = 1000


def _tc_body(x_ref, s1_ref, s2_ref, w_ref, b_ref, o_ref):
    s1 = s1_ref[...]
    s2 = s2_ref[...]
    agg = s2 / (s1 + 1e-16)
    aggf = jnp.concatenate([agg[0], agg[1]], axis=-1)
    feats = x_ref[...] + aggf
    o_ref[...] = (jnp.dot(feats, w_ref[...], preferred_element_type=jnp.float32)
                  + b_ref[0])


def _tc_finish(x, s1, s2, W, b):
    grid = (N // _BR,)
    return pl.pallas_call(
        _tc_body,
        grid=grid,
        in_specs=[
            pl.BlockSpec((_BR, D), lambda i: (i, 0)),
            pl.BlockSpec((NUM_SC, _BR, H), lambda i: (0, i, 0)),
            pl.BlockSpec((NUM_SC, _BR, H), lambda i: (0, i, 0)),
            pl.BlockSpec((D, D), lambda i: (0, 0)),
            pl.BlockSpec((1, D), lambda i: (0, 0)),
        ],
        out_specs=pl.BlockSpec((_BR, D), lambda i: (i, 0)),
        out_shape=jax.ShapeDtypeStruct((N, D), jnp.float32),
    )(x, s1, s2, W, b.reshape(1, D))


def kernel(x, edge_index, edge_attr, bond_tables, W, b):
    src = edge_index[0]
    dst = edge_index[1]
    idx_all = jnp.stack(
        [src.reshape(NCHUNKS, CHUNK), dst.reshape(NCHUNKS, CHUNK)]
        + [edge_attr[:, f].reshape(NCHUNKS, CHUNK) for f in range(4)],
        axis=1)
    t0, t1, t2, t3 = bond_tables[0], bond_tables[1], bond_tables[2], bond_tables[3]
    comb = (t0[:, None, None, None, :] + t1[None, :, None, None, :]
            + t2[None, None, :, None, :] + t3[None, None, None, :, :]
            ).reshape(8 * 8 * 8 * 8, D)
    xr = x.reshape(2 * N, H)
    comb2 = comb.reshape(2 * 8 * 8 * 8 * 8, H)
    s1, s2 = _sc_segment_sums(xr, comb2, idx_all)
    return _tc_finish(x, s1, s2, W, b)


# speedup vs baseline: 2.6313x; 1.0746x over previous
import functools

import jax
import jax.numpy as jnp
import numpy as np
from jax import lax
from jax.experimental import pallas as pl
from jax.experimental.pallas import tpu as pltpu
from jax.experimental.pallas import tpu_sc as plsc

N = 10000
E = 320000
D = 128
H = D // 2
BETA = 1.0
EPS = 1e-07

NUM_SC = 2
NUM_SUB = 16
CHUNK = 128
NCHUNKS = E // CHUNK
N_PAD = 10240
ROWS_PER_SUB = N_PAD // NUM_SUB
DUMP_ROWS = 128
DUMPS_PER_SUB = ROWS_PER_SUB // DUMP_ROWS


def _sc_segment_sums(xr, comb2, idx_all):
    mesh = plsc.VectorSubcoreMesh(core_axis_name="c", subcore_axis_name="s")

    @functools.partial(
        pl.kernel,
        mesh=mesh,
        compiler_params=pltpu.CompilerParams(use_tc_tiling_on_sc=False),
        out_type=[
            jax.ShapeDtypeStruct((NUM_SC, N_PAD, H), jnp.float32),
            jax.ShapeDtypeStruct((NUM_SC, N_PAD, H), jnp.float32),
        ],
        scratch_types=[
            pltpu.VMEM((2, 6, CHUNK), jnp.int32),
            pltpu.VMEM((2, 3, CHUNK), jnp.int32),
            pltpu.VMEM((2, CHUNK, H), jnp.float32),
            pltpu.VMEM((2, CHUNK, H), jnp.float32),
            pltpu.VMEM_SHARED((N_PAD, H), jnp.float32),
            pltpu.VMEM_SHARED((N_PAD, H), jnp.float32),
            pltpu.SemaphoreType.DMA,
            pltpu.SemaphoreType.DMA,
            pltpu.SemaphoreType.DMA,
            pltpu.SemaphoreType.DMA,
            pltpu.SemaphoreType.DMA,
            pltpu.SemaphoreType.DMA,
        ],
    )
    def k(xr_hbm, comb_hbm, idx_hbm,
          s1_out, s2_out,
          idxb, gidx, xrows, crows, s1_sp, s2_sp,
          idxsem0, idxsem1, gsem0, gsem1, scatsem0, scatsem1):
        c = lax.axis_index("c")
        s = lax.axis_index("s")
        idxsem = (idxsem0, idxsem1)
        gsem = (gsem0, gsem1)
        scatsem = (scatsem0, scatsem1)

        def zrow(r, carry):
            for g in range(H // 16):
                xrows[0, r, pl.ds(16 * g, 16)] = jnp.zeros((16,), jnp.float32)
            return carry
        lax.fori_loop(0, DUMP_ROWS, zrow, 0)

        def zstripe(k, carry):
            off = s * ROWS_PER_SUB + k * DUMP_ROWS
            pltpu.sync_copy(xrows.at[0], s1_sp.at[pl.ds(off, DUMP_ROWS)])
            pltpu.sync_copy(xrows.at[0], s2_sp.at[pl.ds(off, DUMP_ROWS)])
            return carry
        lax.fori_loop(0, DUMPS_PER_SUB, zstripe, 0)
        plsc.subcore_barrier()

        NUNIF = (NCHUNKS // NUM_SUB) & ~1

        def idx_start(b, j):
            return pltpu.async_copy(idx_hbm.at[s + j * NUM_SUB], idxb.at[b],
                                    idxsem[b])

        def idx_wait(b):
            pltpu.make_async_copy(idx_hbm.at[s], idxb.at[b], idxsem[b]).wait()

        def idxcomp(b):
            for g in range(CHUNK // 16):
                sl = pl.ds(16 * g, 16)
                cidx = ((idxb[b, 2, sl] * 8 + idxb[b, 3, sl]) * 8
                        + idxb[b, 4, sl]) * 8 + idxb[b, 5, sl]
                gidx[b, 1, sl] = cidx * 2 + c
                gidx[b, 0, sl] = idxb[b, 0, sl] * 2 + c
                gidx[b, 2, sl] = idxb[b, 1, sl]

        def gather_start(b):
            pltpu.async_copy(xr_hbm.at[gidx.at[b, 0]], xrows.at[b], gsem[b])
            pltpu.async_copy(comb_hbm.at[gidx.at[b, 1]], crows.at[b], gsem[b])

        def gather_wait(b):
            pltpu.make_async_copy(xr_hbm.at[gidx.at[b, 0]], xrows.at[b],
                                  gsem[b]).wait()
            pltpu.make_async_copy(comb_hbm.at[gidx.at[b, 1]], crows.at[b],
                                  gsem[b]).wait()

        def compute(b):
            def row(ri, inner):
                for u in range(4):
                    r = 4 * ri + u
                    for g in range(H // 16):
                        sl = pl.ds(16 * g, 16)
                        m = (jnp.maximum(xrows[b, r, sl] + crows[b, r, sl], 0.0)
                             + EPS)
                        ex = jnp.exp(BETA * m)
                        xrows[b, r, sl] = ex
                        crows[b, r, sl] = m * ex
                return inner
            lax.fori_loop(0, CHUNK // 4, row, 0)

        def scat_start(b):
            pltpu.async_copy(xrows.at[b], s1_sp.at[gidx.at[b, 2]], scatsem[b],
                             add=True)
            pltpu.async_copy(crows.at[b], s2_sp.at[gidx.at[b, 2]], scatsem[b],
                             add=True)

        def scat_wait(b):
            pltpu.make_async_copy(xrows.at[b], s1_sp.at[gidx.at[b, 2]],
                                  scatsem[b]).wait()
            pltpu.make_async_copy(crows.at[b], s2_sp.at[gidx.at[b, 2]],
                                  scatsem[b]).wait()

        def process_sync(b):
            compute(b)
            pltpu.sync_copy(xrows.at[b], s1_sp.at[gidx.at[b, 2]], add=True)
            pltpu.sync_copy(crows.at[b], s2_sp.at[gidx.at[b, 2]], add=True)

        idx_start(0, 0)
        idx_start(1, 1)

        def pair(t, carry):
            for b in (0, 1):
                j = 2 * t + b
                idx_wait(b)
                @pl.when(t > 0)
                def _():
                    scat_wait(b)
                idxcomp(b)

                @pl.when(t < NUNIF // 2 - 1)
                def _():
                    idx_start(b, j + 2)
                gather_start(b)
                b1 = 1 - b
                if b == 0:
                    @pl.when(t > 0)
                    def _():
                        gather_wait(b1)
                        compute(b1)
                        scat_start(b1)
                else:
                    gather_wait(b1)
                    compute(b1)
                    scat_start(b1)
            return carry
        lax.fori_loop(0, NUNIF // 2, pair, 0)
        gather_wait(1)
        process_sync(1)
        scat_wait(0)

        ntail = NCHUNKS - NUNIF * NUM_SUB

        @pl.when(s < ntail)
        def _():
            pltpu.sync_copy(idx_hbm.at[NUNIF * NUM_SUB + s], idxb.at[0])
            idxcomp(0)
            gather_start(0)
            gather_wait(0)
            process_sync(0)

        plsc.subcore_barrier()

        def dstripe(k, carry):
            off = s * ROWS_PER_SUB + k * DUMP_ROWS
            pltpu.sync_copy(s1_sp.at[pl.ds(off, DUMP_ROWS)], xrows.at[0])
            pltpu.sync_copy(xrows.at[0], s1_out.at[c, pl.ds(off, DUMP_ROWS)])
            pltpu.sync_copy(s2_sp.at[pl.ds(off, DUMP_ROWS)], crows.at[0])
            pltpu.sync_copy(crows.at[0], s2_out.at[c, pl.ds(off, DUMP_ROWS)])
            return carry
        lax.fori_loop(0, DUMPS_PER_SUB, dstripe, 0)

    return k(xr, comb2, idx_all)


_BR = 1000


def _tc_body(x_ref, s1_ref, s2_ref, w_ref, b_ref, o_ref):
    s1 = s1_ref[...]
    s2 = s2_ref[...]
    agg = s2 / (s1 + 1e-16)
    aggf = jnp.concatenate([agg[0], agg[1]], axis=-1)
    feats = x_ref[...] + aggf
    o_ref[...] = (jnp.dot(feats, w_ref[...], preferred_element_type=jnp.float32)
                  + b_ref[0])


def _tc_finish(x, s1, s2, W, b):
    grid = (N // _BR,)
    return pl.pallas_call(
        _tc_body,
        grid=grid,
        in_specs=[
            pl.BlockSpec((_BR, D), lambda i: (i, 0)),
            pl.BlockSpec((NUM_SC, _BR, H), lambda i: (0, i, 0)),
            pl.BlockSpec((NUM_SC, _BR, H), lambda i: (0, i, 0)),
            pl.BlockSpec((D, D), lambda i: (0, 0)),
            pl.BlockSpec((1, D), lambda i: (0, 0)),
        ],
        out_specs=pl.BlockSpec((_BR, D), lambda i: (i, 0)),
        out_shape=jax.ShapeDtypeStruct((N, D), jnp.float32),
    )(x, s1, s2, W, b.reshape(1, D))


def kernel(x, edge_index, edge_attr, bond_tables, W, b):
    src = edge_index[0]
    dst = edge_index[1]
    idx_all = jnp.stack(
        [src.reshape(NCHUNKS, CHUNK), dst.reshape(NCHUNKS, CHUNK)]
        + [edge_attr[:, f].reshape(NCHUNKS, CHUNK) for f in range(4)],
        axis=1)
    t0, t1, t2, t3 = bond_tables[0], bond_tables[1], bond_tables[2], bond_tables[3]
    comb = (t0[:, None, None, None, :] + t1[None, :, None, None, :]
            + t2[None, None, :, None, :] + t3[None, None, None, :, :]
            ).reshape(8 * 8 * 8 * 8, D)
    xr = x.reshape(2 * N, H)
    comb2 = comb.reshape(2 * 8 * 8 * 8 * 8, H)
    s1, s2 = _sc_segment_sums(xr, comb2, idx_all)
    return _tc_finish(x, s1, s2, W, b)


# speedup vs baseline: 2.7106x; 1.0301x over previous
import functools

import jax
import jax.numpy as jnp
import numpy as np
from jax import lax
from jax.experimental import pallas as pl
from jax.experimental.pallas import tpu as pltpu
from jax.experimental.pallas import tpu_sc as plsc

N = 10000
E = 320000
D = 128
H = D // 2
BETA = 1.0
EPS = 1e-07

NUM_SC = 2
NUM_SUB = 16
CHUNK = 128
NCHUNKS = E // CHUNK
N_PAD = 10240
ROWS_PER_SUB = N_PAD // NUM_SUB
DUMP_ROWS = 128
DUMPS_PER_SUB = ROWS_PER_SUB // DUMP_ROWS


def _sc_segment_sums(xr, comb2, idx_all):
    mesh = plsc.VectorSubcoreMesh(core_axis_name="c", subcore_axis_name="s")

    @functools.partial(
        pl.kernel,
        mesh=mesh,
        compiler_params=pltpu.CompilerParams(use_tc_tiling_on_sc=False),
        out_type=[
            jax.ShapeDtypeStruct((NUM_SC, N_PAD, H), jnp.float32),
            jax.ShapeDtypeStruct((NUM_SC, N_PAD, H), jnp.float32),
        ],
        scratch_types=[
            pltpu.VMEM((2, 6, CHUNK), jnp.int32),
            pltpu.VMEM((2, 3, CHUNK), jnp.int32),
            pltpu.VMEM((2, CHUNK, H), jnp.float32),
            pltpu.VMEM((2, CHUNK, H), jnp.float32),
            pltpu.VMEM_SHARED((N_PAD, H), jnp.float32),
            pltpu.VMEM_SHARED((N_PAD, H), jnp.float32),
            pltpu.SemaphoreType.DMA,
            pltpu.SemaphoreType.DMA,
            pltpu.SemaphoreType.DMA,
            pltpu.SemaphoreType.DMA,
            pltpu.SemaphoreType.DMA,
            pltpu.SemaphoreType.DMA,
        ],
    )
    def k(xr_hbm, comb_hbm, idx_hbm,
          s1_out, s2_out,
          idxb, gidx, xrows, crows, s1_sp, s2_sp,
          idxsem0, idxsem1, gsem0, gsem1, scatsem0, scatsem1):
        c = lax.axis_index("c")
        s = lax.axis_index("s")
        idxsem = (idxsem0, idxsem1)
        gsem = (gsem0, gsem1)
        scatsem = (scatsem0, scatsem1)

        def zrow(r, carry):
            for g in range(H // 16):
                xrows[0, r, pl.ds(16 * g, 16)] = jnp.zeros((16,), jnp.float32)
            return carry
        lax.fori_loop(0, DUMP_ROWS, zrow, 0)

        def zstripe(k, carry):
            off = s * ROWS_PER_SUB + k * DUMP_ROWS
            pltpu.sync_copy(xrows.at[0], s1_sp.at[pl.ds(off, DUMP_ROWS)])
            pltpu.sync_copy(xrows.at[0], s2_sp.at[pl.ds(off, DUMP_ROWS)])
            return carry
        lax.fori_loop(0, DUMPS_PER_SUB, zstripe, 0)
        plsc.subcore_barrier()

        NUNIF = (NCHUNKS // NUM_SUB) & ~1

        def idx_start(b, j):
            return pltpu.async_copy(idx_hbm.at[s + j * NUM_SUB], idxb.at[b],
                                    idxsem[b])

        def idx_wait(b):
            pltpu.make_async_copy(idx_hbm.at[s], idxb.at[b], idxsem[b]).wait()

        def idxcomp(b):
            for g in range(CHUNK // 16):
                sl = pl.ds(16 * g, 16)
                cidx = ((idxb[b, 2, sl] * 8 + idxb[b, 3, sl]) * 8
                        + idxb[b, 4, sl]) * 8 + idxb[b, 5, sl]
                gidx[b, 1, sl] = cidx * 2 + c
                gidx[b, 0, sl] = idxb[b, 0, sl] * 2 + c
                gidx[b, 2, sl] = idxb[b, 1, sl]

        def gather_start(b):
            pltpu.async_copy(xr_hbm.at[gidx.at[b, 0]], xrows.at[b], gsem[b])
            pltpu.async_copy(comb_hbm.at[gidx.at[b, 1]], crows.at[b], gsem[b])

        def gather_wait(b):
            pltpu.make_async_copy(xr_hbm.at[gidx.at[b, 0]], xrows.at[b],
                                  gsem[b]).wait()
            pltpu.make_async_copy(comb_hbm.at[gidx.at[b, 1]], crows.at[b],
                                  gsem[b]).wait()

        def compute(b):
            def row(ri, inner):
                for u in range(8):
                    r = 8 * ri + u
                    for g in range(H // 16):
                        sl = pl.ds(16 * g, 16)
                        m = (jnp.maximum(xrows[b, r, sl] + crows[b, r, sl], 0.0)
                             + EPS)
                        ex = jnp.exp(BETA * m)
                        xrows[b, r, sl] = ex
                        crows[b, r, sl] = m * ex
                return inner
            lax.fori_loop(0, CHUNK // 8, row, 0)

        def scat_start(b):
            pltpu.async_copy(xrows.at[b], s1_sp.at[gidx.at[b, 2]], scatsem[b],
                             add=True)
            pltpu.async_copy(crows.at[b], s2_sp.at[gidx.at[b, 2]], scatsem[b],
                             add=True)

        def scat_wait(b):
            pltpu.make_async_copy(xrows.at[b], s1_sp.at[gidx.at[b, 2]],
                                  scatsem[b]).wait()
            pltpu.make_async_copy(crows.at[b], s2_sp.at[gidx.at[b, 2]],
                                  scatsem[b]).wait()

        def process_sync(b):
            compute(b)
            pltpu.sync_copy(xrows.at[b], s1_sp.at[gidx.at[b, 2]], add=True)
            pltpu.sync_copy(crows.at[b], s2_sp.at[gidx.at[b, 2]], add=True)

        idx_start(0, 0)
        idx_start(1, 1)

        def pair(t, carry):
            for b in (0, 1):
                j = 2 * t + b
                idx_wait(b)
                @pl.when(t > 0)
                def _():
                    scat_wait(b)
                idxcomp(b)

                @pl.when(t < NUNIF // 2 - 1)
                def _():
                    idx_start(b, j + 2)
                gather_start(b)
                b1 = 1 - b
                if b == 0:
                    @pl.when(t > 0)
                    def _():
                        gather_wait(b1)
                        compute(b1)
                        scat_start(b1)
                else:
                    gather_wait(b1)
                    compute(b1)
                    scat_start(b1)
            return carry
        lax.fori_loop(0, NUNIF // 2, pair, 0)
        gather_wait(1)
        process_sync(1)
        scat_wait(0)

        ntail = NCHUNKS - NUNIF * NUM_SUB

        @pl.when(s < ntail)
        def _():
            pltpu.sync_copy(idx_hbm.at[NUNIF * NUM_SUB + s], idxb.at[0])
            idxcomp(0)
            gather_start(0)
            gather_wait(0)
            process_sync(0)

        plsc.subcore_barrier()

        def dstripe(k, carry):
            off = s * ROWS_PER_SUB + k * DUMP_ROWS
            pltpu.sync_copy(s1_sp.at[pl.ds(off, DUMP_ROWS)], xrows.at[0])
            pltpu.sync_copy(xrows.at[0], s1_out.at[c, pl.ds(off, DUMP_ROWS)])
            pltpu.sync_copy(s2_sp.at[pl.ds(off, DUMP_ROWS)], crows.at[0])
            pltpu.sync_copy(crows.at[0], s2_out.at[c, pl.ds(off, DUMP_ROWS)])
            return carry
        lax.fori_loop(0, DUMPS_PER_SUB, dstripe, 0)

    return k(xr, comb2, idx_all)


_BR = 1000


def _tc_body(x_ref, s1_ref, s2_ref, w_ref, b_ref, o_ref):
    s1 = s1_ref[...]
    s2 = s2_ref[...]
    agg = s2 / (s1 + 1e-16)
    aggf = jnp.concatenate([agg[0], agg[1]], axis=-1)
    feats = x_ref[...] + aggf
    o_ref[...] = (jnp.dot(feats, w_ref[...], preferred_element_type=jnp.float32)
                  + b_ref[0])


def _tc_finish(x, s1, s2, W, b):
    grid = (N // _BR,)
    return pl.pallas_call(
        _tc_body,
        grid=grid,
        in_specs=[
            pl.BlockSpec((_BR, D), lambda i: (i, 0)),
            pl.BlockSpec((NUM_SC, _BR, H), lambda i: (0, i, 0)),
            pl.BlockSpec((NUM_SC, _BR, H), lambda i: (0, i, 0)),
            pl.BlockSpec((D, D), lambda i: (0, 0)),
            pl.BlockSpec((1, D), lambda i: (0, 0)),
        ],
        out_specs=pl.BlockSpec((_BR, D), lambda i: (i, 0)),
        out_shape=jax.ShapeDtypeStruct((N, D), jnp.float32),
    )(x, s1, s2, W, b.reshape(1, D))


def kernel(x, edge_index, edge_attr, bond_tables, W, b):
    src = edge_index[0]
    dst = edge_index[1]
    idx_all = jnp.stack(
        [src.reshape(NCHUNKS, CHUNK), dst.reshape(NCHUNKS, CHUNK)]
        + [edge_attr[:, f].reshape(NCHUNKS, CHUNK) for f in range(4)],
        axis=1)
    t0, t1, t2, t3 = bond_tables[0], bond_tables[1], bond_tables[2], bond_tables[3]
    comb = (t0[:, None, None, None, :] + t1[None, :, None, None, :]
            + t2[None, None, :, None, :] + t3[None, None, None, :, :]
            ).reshape(8 * 8 * 8 * 8, D)
    xr = x.reshape(2 * N, H)
    comb2 = comb.reshape(2 * 8 * 8 * 8 * 8, H)
    s1, s2 = _sc_segment_sums(xr, comb2, idx_all)
    return _tc_finish(x, s1, s2, W, b)
